# baseline (speedup 1.0000x reference)
import functools

import jax
import jax.numpy as jnp
from jax import lax
from jax.experimental import pallas as pl
from jax.experimental.pallas import tpu as pltpu
from jax.experimental.pallas import tpu_sc as plsc

N = 10000
E = 320000
IN_FEATS = 128
HID = 64
HH = HID // 2
OUT = 64
G = 128

NC = 2
NS = 16
NW = NC * NS
CH = 128
EPT = 20480
NCHT = EPT // CH
EP = NS * EPT
NP = 10240
RPT = NP // NS

BCR = 4
BC = BCR * CH
NSUP = NCHT // BCR

_mesh = plsc.VectorSubcoreMesh(core_axis_name="c", subcore_axis_name="s")
_sc_params = pltpu.CompilerParams(use_tc_tiling_on_sc=False)


@functools.partial(
    pl.kernel,
    out_type=jax.ShapeDtypeStruct((NC, NP, 16), jnp.float32),
    mesh=_mesh,
    scratch_types=[
        pltpu.VMEM((NCHT // 2, CH), jnp.int32),
        pltpu.VMEM((CH, 16), jnp.float32),
        pltpu.VMEM_SHARED((NP, 16), jnp.float32),
        pltpu.SemaphoreType.DMA,
    ],
    compiler_params=_sc_params,
)
def _deg_pass(didx_hbm, zeros_hbm, out_hbm, didx_v, ones_v, acc_sh, sem):
    c = lax.axis_index("c")
    s = lax.axis_index("s")
    wid = s * NC + c
    pltpu.sync_copy(zeros_hbm.at[pl.ds(s * RPT, RPT)], acc_sh.at[pl.ds(s * RPT, RPT)])
    pltpu.sync_copy(didx_hbm.at[wid], didx_v)

    def fill(r, carry):
        ones_v[r, :] = jnp.ones((16,), jnp.float32)
        return carry

    lax.fori_loop(0, CH, fill, 0)
    plsc.subcore_barrier()

    def chunk(j, carry):
        pltpu.async_copy(ones_v, acc_sh.at[didx_v.at[j]], sem, add=True)
        return carry

    lax.fori_loop(0, NCHT // 2, chunk, 0)

    def drain(j, carry):
        pltpu.make_async_copy(zeros_hbm.at[pl.ds(0, CH)], ones_v, sem).wait()
        return carry

    lax.fori_loop(0, NCHT // 2, drain, 0)
    plsc.subcore_barrier()
    pltpu.sync_copy(acc_sh.at[pl.ds(s * RPT, RPT)], out_hbm.at[c, pl.ds(s * RPT, RPT)])


@functools.partial(
    pl.kernel,
    out_type=jax.ShapeDtypeStruct((NP, HID), jnp.float32),
    mesh=_mesh,
    scratch_types=[
        pltpu.VMEM((NSUP, BC), jnp.int32),
        pltpu.VMEM((NSUP, BC), jnp.int32),
        pltpu.VMEM((2, BC, HH), jnp.float32),
        pltpu.VMEM_SHARED((NP, HH), jnp.float32),
        pltpu.VMEM_SHARED((NP, HH), jnp.float32),
        pltpu.SemaphoreType.DMA((2,)),
        pltpu.SemaphoreType.DMA((2,)),
    ],
    compiler_params=_sc_params,
)
def _edge_pass(g_hbm, sidx_hbm, didx_hbm, zeros_hbm, out_hbm,
               sidx_v, didx_v, rows_v, gtab_sh, acc_sh, gsem, ssem):
    c = lax.axis_index("c")
    s = lax.axis_index("s")
    pltpu.sync_copy(zeros_hbm.at[pl.ds(s * RPT, RPT)], acc_sh.at[pl.ds(s * RPT, RPT)])
    pltpu.sync_copy(g_hbm.at[pl.ds(s * RPT, RPT), pl.ds(c * HH, HH)],
                    gtab_sh.at[pl.ds(s * RPT, RPT)])
    pltpu.sync_copy(sidx_hbm.at[s], sidx_v)
    pltpu.sync_copy(didx_hbm.at[s], didx_v)
    plsc.subcore_barrier()


    def fire_gather(t):
        p = t % 2
        pltpu.async_copy(gtab_sh.at[sidx_v.at[t]],
                         rows_v.at[p], gsem.at[p])

    def drain(buf, sem):
        pltpu.make_async_copy(zeros_hbm.at[pl.ds(0, BC)],
                              rows_v.at[buf], sem).wait()

    fire_gather(0)

    def body(t, carry):
        p = t % 2
        q = (t + 1) % 2

        @pl.when(t >= 1)
        def _():
            drain(q, ssem.at[q])

        @pl.when(t + 1 < NSUP)
        def _():
            fire_gather(t + 1)

        drain(p, gsem.at[p])
        pltpu.async_copy(rows_v.at[p],
                         acc_sh.at[didx_v.at[t]],
                         ssem.at[p], add=True)
        return carry

    lax.fori_loop(0, NSUP, body, 0)
    drain((NSUP - 1) % 2, ssem.at[(NSUP - 1) % 2])
    plsc.subcore_barrier()
    pltpu.sync_copy(acc_sh.at[pl.ds(s * RPT, RPT)],
                    out_hbm.at[pl.ds(s * RPT, RPT), pl.ds(c * HH, HH)])


def _tc_a(deg_ref, x_ref, w1_ref, g1_ref, dinv_ref):
    deg = (deg_ref[0] + deg_ref[1])[:N, 0:1]
    dinv = lax.rsqrt(deg + 1.0)
    h1 = jnp.dot(x_ref[...], w1_ref[...], preferred_element_type=jnp.float32)
    g1_ref[pl.ds(0, N), :] = h1 * dinv
    g1_ref[pl.ds(N, NP - N), :] = jnp.zeros((NP - N, HID), jnp.float32)
    dinv_ref[...] = dinv


def _tc_b(accp_ref, g1_ref, dinv_ref, b1_ref, root_ref, batch_ref, w2_ref,
          g2_ref, x2_ref):
    acc = accp_ref[pl.ds(0, N), :] + g1_ref[pl.ds(0, N), :]
    dinv = dinv_ref[...]
    x2 = dinv * acc + b1_ref[...]
    oh = (batch_ref[...] == lax.broadcasted_iota(jnp.int32, (N, G), 1)
          ).astype(jnp.float32)
    rext = jnp.dot(oh, root_ref[...], preferred_element_type=jnp.float32)
    hcat = jnp.maximum(jnp.concatenate([x2, rext], axis=1), 0.0)
    gg = jnp.dot(hcat, w2_ref[...], preferred_element_type=jnp.float32) * dinv
    g2_ref[pl.ds(0, N), :] = gg
    g2_ref[pl.ds(N, NP - N), :] = jnp.zeros((NP - N, OUT), jnp.float32)
    x2_ref[...] = x2


def _tc_c(accp_ref, g2_ref, dinv_ref, b2_ref, x2_ref, batch_ref, ridx_ref,
          out_ref):
    acc = accp_ref[pl.ds(0, N), :] + g2_ref[pl.ds(0, N), :]
    dinv = dinv_ref[...]
    h3 = jnp.maximum(dinv * acc + b2_ref[...], 0.0)
    ohr = (ridx_ref[...] == lax.broadcasted_iota(jnp.int32, (G, N), 1)
           ).astype(jnp.float32)
    xroot = jnp.dot(ohr, x2_ref[...], preferred_element_type=jnp.float32)
    oh = (batch_ref[...] == lax.broadcasted_iota(jnp.int32, (N, G), 1)
          ).astype(jnp.float32)
    re2 = jnp.dot(oh, xroot, preferred_element_type=jnp.float32)
    hf = jnp.concatenate([h3, re2], axis=1)
    seg = lax.dot_general(oh, hf, (((0,), (0,)), ((), ())),
                          preferred_element_type=jnp.float32)
    counts = lax.dot_general(oh, jnp.ones((N, 1), jnp.float32),
                             (((0,), (0,)), ((), ())),
                             preferred_element_type=jnp.float32)
    out_ref[...] = seg / jnp.maximum(counts, 1.0)


def kernel(x, edge_index, root, batch, rootindex, W1, b1, W2, b2):
    src = edge_index[0]
    dst = edge_index[1]
    pad = jnp.full((EP - E,), N, dtype=jnp.int32)
    srcp = jnp.concatenate([src, pad]).reshape(NS, NSUP, BC)
    dstp = jnp.concatenate([dst, pad]).reshape(NS, NSUP, BC)
    dstw = dstp.reshape(NW, NCHT // 2, CH)
    z16 = jnp.zeros((NP, 16), jnp.float32)
    z32 = jnp.zeros((NP, HH), jnp.float32)
    batch2 = batch.reshape(N, 1)
    ridx2 = rootindex.reshape(G, 1)

    degp = _deg_pass(dstw, z16)

    g1, dinv = pl.pallas_call(
        _tc_a,
        out_shape=(
            jax.ShapeDtypeStruct((NP, HID), jnp.float32),
            jax.ShapeDtypeStruct((N, 1), jnp.float32),
        ),
    )(degp, x, W1)

    accp1 = _edge_pass(g1, srcp, dstp, z32)

    g2, x2 = pl.pallas_call(
        _tc_b,
        out_shape=(
            jax.ShapeDtypeStruct((NP, OUT), jnp.float32),
            jax.ShapeDtypeStruct((N, HID), jnp.float32),
        ),
    )(accp1, g1, dinv, b1.reshape(1, HID), root, batch2, W2)

    accp2 = _edge_pass(g2, srcp, dstp, z32)

    out = pl.pallas_call(
        _tc_c,
        out_shape=jax.ShapeDtypeStruct((G, IN_FEATS), jnp.float32),
    )(accp2, g2, dinv, b2.reshape(1, OUT), x2, batch2, ridx2)

    return out


# speedup vs baseline: 1.0160x; 1.0160x over previous
import functools

import jax
import jax.numpy as jnp
from jax import lax
from jax.experimental import pallas as pl
from jax.experimental.pallas import tpu as pltpu
from jax.experimental.pallas import tpu_sc as plsc

N = 10000
E = 320000
IN_FEATS = 128
HID = 64
HH = HID // 2
OUT = 64
G = 128

NC = 2
NS = 16
NW = NC * NS
CH = 128
EPT = 20480
NCHT = EPT // CH
EP = NS * EPT
NP = 10240
RPT = NP // NS

BCR = 4
BC = BCR * CH
NSUP = NCHT // BCR

_mesh = plsc.VectorSubcoreMesh(core_axis_name="c", subcore_axis_name="s")
_sc_params = pltpu.CompilerParams(use_tc_tiling_on_sc=False)


NDC = NSUP // 2


@functools.partial(
    pl.kernel,
    out_type=jax.ShapeDtypeStruct((NC, NP, 16), jnp.float32),
    mesh=_mesh,
    scratch_types=[
        pltpu.VMEM((NDC, BC), jnp.int32),
        pltpu.VMEM((BC, 16), jnp.float32),
        pltpu.VMEM_SHARED((NP, 16), jnp.float32),
        pltpu.SemaphoreType.DMA,
    ],
    compiler_params=_sc_params,
)
def _deg_pass(ei_hbm, zeros_hbm, out_hbm, didx_v, ones_v, acc_sh, sem):
    c = lax.axis_index("c")
    s = lax.axis_index("s")
    pltpu.sync_copy(zeros_hbm.at[pl.ds(s * RPT, RPT)], acc_sh.at[pl.ds(s * RPT, RPT)])
    pltpu.sync_copy(ei_hbm.at[1, s, pl.ds(c * NDC, NDC)], didx_v)

    def fill(r, carry):
        ones_v[r, :] = jnp.ones((16,), jnp.float32)
        return carry

    lax.fori_loop(0, BC, fill, 0)
    plsc.subcore_barrier()

    def chunk(j, carry):
        pltpu.async_copy(ones_v, acc_sh.at[didx_v.at[j]], sem, add=True)
        return carry

    lax.fori_loop(0, NDC, chunk, 0)

    def drain(j, carry):
        pltpu.make_async_copy(zeros_hbm.at[pl.ds(0, BC)], ones_v, sem).wait()
        return carry

    lax.fori_loop(0, NDC, drain, 0)
    plsc.subcore_barrier()
    pltpu.sync_copy(acc_sh.at[pl.ds(s * RPT, RPT)], out_hbm.at[c, pl.ds(s * RPT, RPT)])


@functools.partial(
    pl.kernel,
    out_type=jax.ShapeDtypeStruct((NP, HID), jnp.float32),
    mesh=_mesh,
    scratch_types=[
        pltpu.VMEM((NSUP, BC), jnp.int32),
        pltpu.VMEM((NSUP, BC), jnp.int32),
        pltpu.VMEM((2, BC, HH), jnp.float32),
        pltpu.VMEM_SHARED((NP, HH), jnp.float32),
        pltpu.VMEM_SHARED((NP, HH), jnp.float32),
        pltpu.SemaphoreType.DMA((2,)),
        pltpu.SemaphoreType.DMA((2,)),
    ],
    compiler_params=_sc_params,
)
def _edge_pass(g_hbm, ei_hbm, zeros_hbm, out_hbm,
               sidx_v, didx_v, rows_v, gtab_sh, acc_sh, gsem, ssem):
    c = lax.axis_index("c")
    s = lax.axis_index("s")
    pltpu.sync_copy(g_hbm.at[pl.ds(s * RPT, RPT), pl.ds(c * HH, HH)],
                    acc_sh.at[pl.ds(s * RPT, RPT)])
    pltpu.sync_copy(g_hbm.at[pl.ds(s * RPT, RPT), pl.ds(c * HH, HH)],
                    gtab_sh.at[pl.ds(s * RPT, RPT)])
    pltpu.sync_copy(ei_hbm.at[0, s], sidx_v)
    pltpu.sync_copy(ei_hbm.at[1, s], didx_v)
    plsc.subcore_barrier()


    def fire_gather(t):
        p = t % 2
        pltpu.async_copy(gtab_sh.at[sidx_v.at[t]],
                         rows_v.at[p], gsem.at[p])

    def drain(buf, sem):
        pltpu.make_async_copy(zeros_hbm.at[pl.ds(0, BC)],
                              rows_v.at[buf], sem).wait()

    fire_gather(0)

    def body(t, carry):
        p = t % 2
        q = (t + 1) % 2

        @pl.when(t >= 1)
        def _():
            drain(q, ssem.at[q])

        @pl.when(t + 1 < NSUP)
        def _():
            fire_gather(t + 1)

        drain(p, gsem.at[p])
        pltpu.async_copy(rows_v.at[p],
                         acc_sh.at[didx_v.at[t]],
                         ssem.at[p], add=True)
        return carry

    lax.fori_loop(0, NSUP, body, 0)
    drain((NSUP - 1) % 2, ssem.at[(NSUP - 1) % 2])
    plsc.subcore_barrier()
    pltpu.sync_copy(acc_sh.at[pl.ds(s * RPT, RPT)],
                    out_hbm.at[pl.ds(s * RPT, RPT), pl.ds(c * HH, HH)])


def _tc_a(deg_ref, x_ref, w1_ref, g1_ref, dinv_ref):
    deg = (deg_ref[0] + deg_ref[1])[:N, 0:1]
    dinv = lax.rsqrt(deg + 1.0)
    h1 = jnp.dot(x_ref[...], w1_ref[...], preferred_element_type=jnp.float32)
    g1_ref[pl.ds(0, N), :] = h1 * dinv
    g1_ref[pl.ds(N, NP - N), :] = jnp.zeros((NP - N, HID), jnp.float32)
    dinv_ref[...] = dinv


def _tc_b0(root_ref, batch_ref, rext_ref):
    oh = (batch_ref[...] == lax.broadcasted_iota(jnp.int32, (N, G), 1)
          ).astype(jnp.float32)
    rext_ref[...] = jnp.dot(oh, root_ref[...],
                            preferred_element_type=jnp.float32)


def _tc_b1(accp_ref, dinv_ref, b1_ref, rext_ref, w2_ref, g2_ref, x2_ref):
    dinv = dinv_ref[...]
    x2 = dinv * accp_ref[pl.ds(0, N), :] + b1_ref[...]
    hcat = jnp.maximum(jnp.concatenate([x2, rext_ref[...]], axis=1), 0.0)
    gg = jnp.dot(hcat, w2_ref[...], preferred_element_type=jnp.float32) * dinv
    g2_ref[pl.ds(0, N), :] = gg
    g2_ref[pl.ds(N, NP - N), :] = jnp.zeros((NP - N, OUT), jnp.float32)
    x2_ref[...] = x2


def _tc_c0(x2_ref, batch_ref, ridx_ref, re2_ref, counts_ref):
    ohr = (ridx_ref[...] == lax.broadcasted_iota(jnp.int32, (G, N), 1)
           ).astype(jnp.float32)
    xroot = jnp.dot(ohr, x2_ref[...], preferred_element_type=jnp.float32)
    oh = (batch_ref[...] == lax.broadcasted_iota(jnp.int32, (N, G), 1)
          ).astype(jnp.float32)
    re2_ref[...] = jnp.dot(oh, xroot, preferred_element_type=jnp.float32)
    counts_ref[...] = lax.dot_general(oh, jnp.ones((N, 1), jnp.float32),
                                      (((0,), (0,)), ((), ())),
                                      preferred_element_type=jnp.float32)


def _tc_c1(accp_ref, dinv_ref, b2_ref, re2_ref, counts_ref, batch_ref,
           out_ref):
    dinv = dinv_ref[...]
    h3 = jnp.maximum(dinv * accp_ref[pl.ds(0, N), :] + b2_ref[...], 0.0)
    hf = jnp.concatenate([h3, re2_ref[...]], axis=1)
    oh = (batch_ref[...] == lax.broadcasted_iota(jnp.int32, (N, G), 1)
          ).astype(jnp.float32)
    seg = lax.dot_general(oh, hf, (((0,), (0,)), ((), ())),
                          preferred_element_type=jnp.float32)
    out_ref[...] = seg / jnp.maximum(counts_ref[...], 1.0)


def kernel(x, edge_index, root, batch, rootindex, W1, b1, W2, b2):
    ei4 = jnp.pad(edge_index, ((0, 0), (0, EP - E)),
                  constant_values=N).reshape(2, NS, NSUP, BC)
    z16 = jnp.zeros((NP, 16), jnp.float32)
    z32 = jnp.zeros((NP, HH), jnp.float32)
    batch2 = batch.reshape(N, 1)
    ridx2 = rootindex.reshape(G, 1)

    degp = _deg_pass(ei4, z16)

    g1, dinv = pl.pallas_call(
        _tc_a,
        out_shape=(
            jax.ShapeDtypeStruct((NP, HID), jnp.float32),
            jax.ShapeDtypeStruct((N, 1), jnp.float32),
        ),
    )(degp, x, W1)

    rext = pl.pallas_call(
        _tc_b0,
        out_shape=jax.ShapeDtypeStruct((N, IN_FEATS), jnp.float32),
    )(root, batch2)

    accp1 = _edge_pass(g1, ei4, z32)

    g2, x2 = pl.pallas_call(
        _tc_b1,
        out_shape=(
            jax.ShapeDtypeStruct((NP, OUT), jnp.float32),
            jax.ShapeDtypeStruct((N, HID), jnp.float32),
        ),
    )(accp1, dinv, b1.reshape(1, HID), rext, W2)

    re2, counts = pl.pallas_call(
        _tc_c0,
        out_shape=(
            jax.ShapeDtypeStruct((N, HID), jnp.float32),
            jax.ShapeDtypeStruct((G, 1), jnp.float32),
        ),
    )(x2, batch2, ridx2)

    accp2 = _edge_pass(g2, ei4, z32)

    out = pl.pallas_call(
        _tc_c1,
        out_shape=jax.ShapeDtypeStruct((G, IN_FEATS), jnp.float32),
    )(accp2, dinv, b2.reshape(1, OUT), re2, counts, batch2)

    return out


# speedup vs baseline: 1.0187x; 1.0027x over previous
import functools

import jax
import jax.numpy as jnp
from jax import lax
from jax.experimental import pallas as pl
from jax.experimental.pallas import tpu as pltpu
from jax.experimental.pallas import tpu_sc as plsc

N = 10000
E = 320000
IN_FEATS = 128
HID = 64
HH = HID // 2
OUT = 64
G = 128

NC = 2
NS = 16
NW = NC * NS
CH = 128
EPT = 20480
NCHT = EPT // CH
EP = NS * EPT
NP = 10240
RPT = NP // NS

BCR = 4
BC = BCR * CH
NSUP = NCHT // BCR

_mesh = plsc.VectorSubcoreMesh(core_axis_name="c", subcore_axis_name="s")
_sc_params = pltpu.CompilerParams(use_tc_tiling_on_sc=False)


NDC = NSUP // 2


@functools.partial(
    pl.kernel,
    out_type=jax.ShapeDtypeStruct((NC, NP, 16), jnp.float32),
    mesh=_mesh,
    scratch_types=[
        pltpu.VMEM((NDC, BC), jnp.int32),
        pltpu.VMEM((BC, 16), jnp.float32),
        pltpu.VMEM_SHARED((NP, 16), jnp.float32),
        pltpu.SemaphoreType.DMA,
    ],
    compiler_params=_sc_params,
)
def _deg_pass(ei_hbm, zeros_hbm, out_hbm, didx_v, ones_v, acc_sh, sem):
    c = lax.axis_index("c")
    s = lax.axis_index("s")
    pltpu.sync_copy(zeros_hbm.at[pl.ds(s * RPT, RPT)], acc_sh.at[pl.ds(s * RPT, RPT)])
    pltpu.sync_copy(ei_hbm.at[1, s, pl.ds(c * NDC, NDC)], didx_v)

    def fill(r, carry):
        ones_v[r, :] = jnp.ones((16,), jnp.float32)
        return carry

    lax.fori_loop(0, BC, fill, 0)
    plsc.subcore_barrier()

    def chunk(j, carry):
        pltpu.async_copy(ones_v, acc_sh.at[didx_v.at[j]], sem, add=True)
        return carry

    lax.fori_loop(0, NDC, chunk, 0)

    def drain(j, carry):
        pltpu.make_async_copy(zeros_hbm.at[pl.ds(0, BC)], ones_v, sem).wait()
        return carry

    lax.fori_loop(0, NDC, drain, 0)
    plsc.subcore_barrier()
    pltpu.sync_copy(acc_sh.at[pl.ds(s * RPT, RPT)], out_hbm.at[c, pl.ds(s * RPT, RPT)])


@functools.partial(
    pl.kernel,
    out_type=jax.ShapeDtypeStruct((NP, HID), jnp.float32),
    mesh=_mesh,
    scratch_types=[
        pltpu.VMEM((NSUP, BC), jnp.int32),
        pltpu.VMEM((NSUP, BC), jnp.int32),
        pltpu.VMEM((2, BC, HH), jnp.float32),
        pltpu.VMEM_SHARED((NP, HH), jnp.float32),
        pltpu.VMEM_SHARED((NP, HH), jnp.float32),
        pltpu.SemaphoreType.DMA((2,)),
        pltpu.SemaphoreType.DMA((2,)),
    ],
    compiler_params=_sc_params,
)
def _edge_pass(g_hbm, ei_hbm, out_hbm,
               sidx_v, didx_v, rows_v, gtab_sh, acc_sh, gsem, ssem):
    c = lax.axis_index("c")
    s = lax.axis_index("s")
    pltpu.sync_copy(g_hbm.at[pl.ds(s * RPT, RPT), pl.ds(c * HH, HH)],
                    acc_sh.at[pl.ds(s * RPT, RPT)])
    pltpu.sync_copy(g_hbm.at[pl.ds(s * RPT, RPT), pl.ds(c * HH, HH)],
                    gtab_sh.at[pl.ds(s * RPT, RPT)])
    pltpu.sync_copy(ei_hbm.at[0, s], sidx_v)
    pltpu.sync_copy(ei_hbm.at[1, s], didx_v)
    plsc.subcore_barrier()


    def fire_gather(t):
        p = t % 2
        pltpu.async_copy(gtab_sh.at[sidx_v.at[t]],
                         rows_v.at[p], gsem.at[p])

    def drain(buf, sem):
        pltpu.make_async_copy(g_hbm.at[pl.ds(0, BC), pl.ds(0, HH)],
                              rows_v.at[buf], sem).wait()

    fire_gather(0)

    def body(t, carry):
        p = t % 2
        q = (t + 1) % 2

        @pl.when(t >= 1)
        def _():
            drain(q, ssem.at[q])

        @pl.when(t + 1 < NSUP)
        def _():
            fire_gather(t + 1)

        drain(p, gsem.at[p])
        pltpu.async_copy(rows_v.at[p],
                         acc_sh.at[didx_v.at[t]],
                         ssem.at[p], add=True)
        return carry

    lax.fori_loop(0, NSUP, body, 0)
    drain((NSUP - 1) % 2, ssem.at[(NSUP - 1) % 2])
    plsc.subcore_barrier()
    pltpu.sync_copy(acc_sh.at[pl.ds(s * RPT, RPT)],
                    out_hbm.at[pl.ds(s * RPT, RPT), pl.ds(c * HH, HH)])


def _tc_a0(x_ref, w1_ref, h1_ref):
    h1_ref[...] = jnp.dot(x_ref[...], w1_ref[...],
                          preferred_element_type=jnp.float32)


def _tc_a1(deg_ref, h1_ref, g1_ref, dinv_ref):
    deg = (deg_ref[0] + deg_ref[1])[:N, 0:1]
    dinv = lax.rsqrt(deg + 1.0)
    g1_ref[pl.ds(0, N), :] = h1_ref[...] * dinv
    g1_ref[pl.ds(N, NP - N), :] = jnp.zeros((NP - N, HID), jnp.float32)
    dinv_ref[...] = dinv


def _tc_b0(root_ref, batch_ref, rext_ref):
    oh = (batch_ref[...] == lax.broadcasted_iota(jnp.int32, (N, G), 1)
          ).astype(jnp.float32)
    rext_ref[...] = jnp.dot(oh, root_ref[...],
                            preferred_element_type=jnp.float32)


def _tc_b1(accp_ref, dinv_ref, b1_ref, rext_ref, w2_ref, g2_ref, x2_ref):
    dinv = dinv_ref[...]
    x2 = dinv * accp_ref[pl.ds(0, N), :] + b1_ref[...]
    hcat = jnp.maximum(jnp.concatenate([x2, rext_ref[...]], axis=1), 0.0)
    gg = jnp.dot(hcat, w2_ref[...], preferred_element_type=jnp.float32) * dinv
    g2_ref[pl.ds(0, N), :] = gg
    g2_ref[pl.ds(N, NP - N), :] = jnp.zeros((NP - N, OUT), jnp.float32)
    x2_ref[...] = x2


def _tc_c0(x2_ref, batch_ref, ridx_ref, re2_ref, counts_ref):
    ohr = (ridx_ref[...] == lax.broadcasted_iota(jnp.int32, (G, N), 1)
           ).astype(jnp.float32)
    xroot = jnp.dot(ohr, x2_ref[...], preferred_element_type=jnp.float32)
    oh = (batch_ref[...] == lax.broadcasted_iota(jnp.int32, (N, G), 1)
          ).astype(jnp.float32)
    re2_ref[...] = jnp.dot(oh, xroot, preferred_element_type=jnp.float32)
    counts_ref[...] = lax.dot_general(oh, jnp.ones((N, 1), jnp.float32),
                                      (((0,), (0,)), ((), ())),
                                      preferred_element_type=jnp.float32)


def _tc_c1(accp_ref, dinv_ref, b2_ref, re2_ref, counts_ref, batch_ref,
           out_ref):
    dinv = dinv_ref[...]
    h3 = jnp.maximum(dinv * accp_ref[pl.ds(0, N), :] + b2_ref[...], 0.0)
    hf = jnp.concatenate([h3, re2_ref[...]], axis=1)
    oh = (batch_ref[...] == lax.broadcasted_iota(jnp.int32, (N, G), 1)
          ).astype(jnp.float32)
    seg = lax.dot_general(oh, hf, (((0,), (0,)), ((), ())),
                          preferred_element_type=jnp.float32)
    out_ref[...] = seg / jnp.maximum(counts_ref[...], 1.0)


def kernel(x, edge_index, root, batch, rootindex, W1, b1, W2, b2):
    ei4 = jnp.pad(edge_index, ((0, 0), (0, EP - E)),
                  constant_values=N).reshape(2, NS, NSUP, BC)
    z16 = jnp.zeros((NP, 16), jnp.float32)
    batch2 = batch.reshape(N, 1)
    ridx2 = rootindex.reshape(G, 1)

    degp = _deg_pass(ei4, z16)

    h1 = pl.pallas_call(
        _tc_a0,
        out_shape=jax.ShapeDtypeStruct((N, HID), jnp.float32),
    )(x, W1)

    g1, dinv = pl.pallas_call(
        _tc_a1,
        out_shape=(
            jax.ShapeDtypeStruct((NP, HID), jnp.float32),
            jax.ShapeDtypeStruct((N, 1), jnp.float32),
        ),
    )(degp, h1)

    rext = pl.pallas_call(
        _tc_b0,
        out_shape=jax.ShapeDtypeStruct((N, IN_FEATS), jnp.float32),
    )(root, batch2)

    accp1 = _edge_pass(g1, ei4)

    g2, x2 = pl.pallas_call(
        _tc_b1,
        out_shape=(
            jax.ShapeDtypeStruct((NP, OUT), jnp.float32),
            jax.ShapeDtypeStruct((N, HID), jnp.float32),
        ),
    )(accp1, dinv, b1.reshape(1, HID), rext, W2)

    re2, counts = pl.pallas_call(
        _tc_c0,
        out_shape=(
            jax.ShapeDtypeStruct((N, HID), jnp.float32),
            jax.ShapeDtypeStruct((G, 1), jnp.float32),
        ),
    )(x2, batch2, ridx2)

    accp2 = _edge_pass(g2, ei4)

    out = pl.pallas_call(
        _tc_c1,
        out_shape=jax.ShapeDtypeStruct((G, IN_FEATS), jnp.float32),
    )(accp2, dinv, b2.reshape(1, OUT), re2, counts, batch2)

    return out
